# baseline (speedup 1.0000x reference)
import jax
import jax.numpy as jnp
import numpy as np
from jax import lax
from jax.experimental import pallas as pl
from jax.experimental.pallas import tpu as pltpu
from jax.experimental.pallas import tpu_sc as plsc

_N = 36
_PERM = tuple(int(v) for v in np.random.RandomState(0).permutation(_N))

_B = 4096
_D = 128
_NC = 2
_NS = 16
_NW = _NC * _NS
_BPW = _B // _NW


def _body(x_hbm, out_hbm):
    c = lax.axis_index("c")
    s = lax.axis_index("s")
    wid = s * _NC + c
    base = wid * _BPW
    for j, pj in enumerate(_PERM):
        pltpu.sync_copy(
            x_hbm.at[pl.ds(base, _BPW), pl.ds(pj, 1), :],
            out_hbm.at[pl.ds(base, _BPW), pl.ds(j, 1), :],
        )


@jax.jit
def kernel(x):
    mesh = plsc.VectorSubcoreMesh(core_axis_name="c", subcore_axis_name="s")
    return pl.kernel(
        _body,
        out_type=jax.ShapeDtypeStruct((_B, _N, _D), x.dtype),
        mesh=mesh,
    )(x)


# speedup vs baseline: 1.0020x; 1.0020x over previous
import jax
import jax.numpy as jnp
import numpy as np
from jax import lax
from jax.experimental import pallas as pl
from jax.experimental.pallas import tpu as pltpu
from jax.experimental.pallas import tpu_sc as plsc

_N = 36
_PERM = tuple(int(v) for v in np.random.RandomState(0).permutation(_N))

_B = 4096
_D = 128
_NC = 2
_NS = 16
_NW = _NC * _NS
_BPW = _B // _NW


def _body(x_hbm, out_hbm, sem):
    c = lax.axis_index("c")
    s = lax.axis_index("s")
    wid = s * _NC + c
    base = wid * _BPW
    copies = []
    for j, pj in enumerate(_PERM):
        cp = pltpu.make_async_copy(
            x_hbm.at[pl.ds(base, _BPW), pl.ds(pj, 1), :],
            out_hbm.at[pl.ds(base, _BPW), pl.ds(j, 1), :],
            sem,
        )
        cp.start()
        copies.append(cp)
    for cp in copies:
        cp.wait()


@jax.jit
def kernel(x):
    mesh = plsc.VectorSubcoreMesh(core_axis_name="c", subcore_axis_name="s")
    return pl.kernel(
        _body,
        out_type=jax.ShapeDtypeStruct((_B, _N, _D), x.dtype),
        mesh=mesh,
        scratch_types=[pltpu.SemaphoreType.DMA],
    )(x)


# speedup vs baseline: 7.0700x; 7.0558x over previous
import jax
import jax.numpy as jnp
import numpy as np
from jax import lax
from jax.experimental import pallas as pl
from jax.experimental.pallas import tpu as pltpu
from jax.experimental.pallas import tpu_sc as plsc

_N = 36
_PERM = np.random.RandomState(0).permutation(_N).astype(np.int32)

_B = 4096
_D = 128
_NC = 2
_NS = 16
_NW = _NC * _NS
_ROWS = _B * _N
_RPW = _ROWS // _NW
_CL = 128
_NCHUNK = _RPW // _CL

_r = np.arange(_ROWS, dtype=np.int64)
_IDX = ((_r // _N) * _N + _PERM[_r % _N]).astype(np.int32)
_IDX3 = jnp.asarray(_IDX.reshape(_NW, _NCHUNK, _CL))


def _body(x_hbm, idx_hbm, out_hbm, idx_v, buf0, buf1, sem0, sem1):
    wid = lax.axis_index("s") * _NC + lax.axis_index("c")
    rbase = wid * _RPW
    pltpu.sync_copy(idx_hbm.at[wid], idx_v)

    bufs = (buf0, buf1)
    sems = (sem0, sem1)

    def start_gather(chunk, b):
        pltpu.async_copy(x_hbm.at[idx_v.at[chunk]], bufs[b], sems[b])

    start_gather(0, 0)
    start_gather(1, 1)

    def step(i, carry):
        g = i * 2
        for b in range(2):
            chunk = g + b
            pltpu.make_async_copy(
                x_hbm.at[idx_v.at[chunk]], bufs[b], sems[b]
            ).wait()
            pltpu.sync_copy(
                bufs[b], out_hbm.at[pl.ds(rbase + chunk * _CL, _CL), :]
            )

            @pl.when(chunk + 2 < _NCHUNK)
            def _():
                start_gather(chunk + 2, b)

        return carry

    lax.fori_loop(0, _NCHUNK // 2, step, 0)


@jax.jit
def kernel(x):
    xr = x.reshape(_ROWS, _D)
    mesh = plsc.VectorSubcoreMesh(core_axis_name="c", subcore_axis_name="s")
    out = pl.kernel(
        _body,
        out_type=jax.ShapeDtypeStruct((_ROWS, _D), x.dtype),
        mesh=mesh,
        scratch_types=[
            pltpu.VMEM((_NCHUNK, _CL), jnp.int32),
            pltpu.VMEM((_CL, _D), jnp.float32),
            pltpu.VMEM((_CL, _D), jnp.float32),
            pltpu.SemaphoreType.DMA,
            pltpu.SemaphoreType.DMA,
        ],
    )(xr, _IDX3)
    return out.reshape(_B, _N, _D)


# speedup vs baseline: 32.7143x; 4.6272x over previous
import jax
import jax.numpy as jnp
import numpy as np
from jax import lax
from jax.experimental import pallas as pl
from jax.experimental.pallas import tpu as pltpu
from jax.experimental.pallas import tpu_sc as plsc

_N = 36
_PERM = tuple(int(v) for v in np.random.RandomState(0).permutation(_N))

_B = 4096
_D = 128
_NC = 2
_NS = 16
_NW = _NC * _NS
_BPW = _B // _NW


def _body(x_hbm, out_hbm, buf0, buf1, sem0, sem1):
    wid = lax.axis_index("s") * _NC + lax.axis_index("c")
    b0 = wid * _BPW
    bufs = (buf0, buf1)
    sems = (sem0, sem1)

    def start_in(j, b):
        pj = _PERM[j]
        pltpu.async_copy(x_hbm.at[pj, pl.ds(b0, _BPW), :], bufs[b], sems[b])

    start_in(0, 0)
    start_in(1, 1)

    for j in range(_N):
        b = j % 2
        pj = _PERM[j]
        pltpu.make_async_copy(
            x_hbm.at[pj, pl.ds(b0, _BPW), :], bufs[b], sems[b]
        ).wait()
        pltpu.sync_copy(bufs[b], out_hbm.at[j, pl.ds(b0, _BPW), :])
        if j + 2 < _N:
            start_in(j + 2, b)


@jax.jit
def kernel(x):
    xt = jnp.transpose(x, (1, 0, 2))
    mesh = plsc.VectorSubcoreMesh(core_axis_name="c", subcore_axis_name="s")
    out_t = pl.kernel(
        _body,
        out_type=jax.ShapeDtypeStruct((_N, _B, _D), x.dtype),
        mesh=mesh,
        scratch_types=[
            pltpu.VMEM((_BPW, _D), jnp.float32),
            pltpu.VMEM((_BPW, _D), jnp.float32),
            pltpu.SemaphoreType.DMA,
            pltpu.SemaphoreType.DMA,
        ],
    )(xt)
    return jnp.transpose(out_t, (1, 0, 2))
